# baseline (speedup 1.0000x reference)
import jax
import jax.numpy as jnp
from jax import lax
from jax.experimental import pallas as pl
from jax.experimental.pallas import tpu as pltpu
from jax.experimental.pallas import tpu_sc as plsc

_BATCH = 1024
_EMB = 64
_VOCAB = 100000

_NUM_WORKERS = 32
_ROWS_PER_WORKER = _BATCH // _NUM_WORKERS

_BATCH_TILE = 32
_TR_TILE = 2048


def _sc_gather_pairs(table2, idx_half):
    mesh = plsc.VectorSubcoreMesh(core_axis_name="c", subcore_axis_name="s")

    @pl.kernel(
        mesh=mesh,
        out_type=jax.ShapeDtypeStruct((_BATCH, 2 * _EMB), table2.dtype),
        scratch_types=[
            pltpu.VMEM((_ROWS_PER_WORKER,), jnp.int32),
            pltpu.VMEM((_ROWS_PER_WORKER, 2 * _EMB), table2.dtype),
            pltpu.SemaphoreType.DMA,
        ],
    )
    def gather_kernel(table_hbm, idx_hbm, out_hbm, idx_v, rows_v, sem):
        wid = lax.axis_index("s") * 2 + lax.axis_index("c")
        base = wid * _ROWS_PER_WORKER
        pltpu.sync_copy(idx_hbm.at[pl.ds(base, _ROWS_PER_WORKER)], idx_v)
        pltpu.async_copy(table_hbm.at[idx_v], rows_v, sem).wait()
        pltpu.sync_copy(rows_v, out_hbm.at[pl.ds(base, _ROWS_PER_WORKER)])

    return gather_kernel(table2, idx_half)


def _transpose_body(w_ref, o_ref):
    o_ref[...] = w_ref[...].astype(jnp.bfloat16).T


def _tc_transpose(W):
    return pl.pallas_call(
        _transpose_body,
        grid=(pl.cdiv(_VOCAB, _TR_TILE),),
        in_specs=[pl.BlockSpec((_TR_TILE, _EMB), lambda j: (j, 0))],
        out_specs=pl.BlockSpec((_EMB, _TR_TILE), lambda j: (0, j)),
        out_shape=jax.ShapeDtypeStruct((_EMB, _VOCAB), jnp.bfloat16),
    )(W)


def _fused_body(e2_ref, p_ref, w_ref, b_ref, o_ref):
    e2 = e2_ref[...]
    par = p_ref[...] == 1
    e = jnp.where(par, e2[:, _EMB:], e2[:, :_EMB]).astype(jnp.bfloat16)
    x = lax.dot_general(
        e, w_ref[...], (((1,), (0,)), ((), ())),
        preferred_element_type=jnp.float32,
    )
    x = x + b_ref[...]
    m = jnp.max(x, axis=1, keepdims=True)
    lse = jnp.log(jnp.sum(jnp.exp(x - m), axis=1, keepdims=True)) + m
    o_ref[...] = x - lse


def _tc_logsoftmax(embeds2, parity, w_t, b2d):
    return pl.pallas_call(
        _fused_body,
        grid=(_BATCH // _BATCH_TILE,),
        in_specs=[
            pl.BlockSpec((_BATCH_TILE, 2 * _EMB), lambda i: (i, 0)),
            pl.BlockSpec((_BATCH_TILE, 1), lambda i: (i, 0)),
            pl.BlockSpec((_EMB, _VOCAB), lambda i: (0, 0)),
            pl.BlockSpec((1, _VOCAB), lambda i: (0, 0)),
        ],
        out_specs=pl.BlockSpec((_BATCH_TILE, _VOCAB), lambda i: (i, 0)),
        out_shape=jax.ShapeDtypeStruct((_BATCH, _VOCAB), jnp.float32),
    )(embeds2, parity, w_t, b2d)


def kernel(input_word, emb_table, W, b):
    idx = input_word.astype(jnp.int32)
    table2 = emb_table.reshape(_VOCAB // 2, 2 * _EMB)
    embeds2 = _sc_gather_pairs(table2, idx >> 1)
    parity = (idx & 1).reshape(_BATCH, 1)
    w_t = _tc_transpose(W)
    return _tc_logsoftmax(embeds2, parity, w_t, b.reshape(1, _VOCAB))


# speedup vs baseline: 1.8130x; 1.8130x over previous
import jax
import jax.numpy as jnp
from jax import lax
from jax.experimental import pallas as pl
from jax.experimental.pallas import tpu as pltpu
from jax.experimental.pallas import tpu_sc as plsc

_BATCH = 1024
_EMB = 64
_VOCAB = 100000

_NUM_WORKERS = 32
_ROWS_PER_WORKER = _BATCH // _NUM_WORKERS

_VT = 2048
_NV = pl.cdiv(_VOCAB, _VT)


def _sc_gather_pairs(table2, idx_half):
    mesh = plsc.VectorSubcoreMesh(core_axis_name="c", subcore_axis_name="s")

    @pl.kernel(
        mesh=mesh,
        out_type=jax.ShapeDtypeStruct((_BATCH, 2 * _EMB), table2.dtype),
        scratch_types=[
            pltpu.VMEM((_ROWS_PER_WORKER,), jnp.int32),
            pltpu.VMEM((_ROWS_PER_WORKER, 2 * _EMB), table2.dtype),
            pltpu.SemaphoreType.DMA,
        ],
    )
    def gather_kernel(table_hbm, idx_hbm, out_hbm, idx_v, rows_v, sem):
        wid = lax.axis_index("s") * 2 + lax.axis_index("c")
        base = wid * _ROWS_PER_WORKER
        pltpu.sync_copy(idx_hbm.at[pl.ds(base, _ROWS_PER_WORKER)], idx_v)
        pltpu.async_copy(table_hbm.at[idx_v], rows_v, sem).wait()
        pltpu.sync_copy(rows_v, out_hbm.at[pl.ds(base, _ROWS_PER_WORKER)])

    return gather_kernel(table2, idx_half)


def _select_augment(e2, par):
    e = jnp.where(par == 1, e2[:, _EMB:], e2[:, :_EMB])
    ones = jnp.ones((_BATCH, 1), jnp.float32)
    return jnp.concatenate([e, ones], axis=1).astype(jnp.bfloat16)


def _lse_body(e2_ref, p_ref, w_ref, o_ref, e_scr, m_scr, s_scr):
    j = pl.program_id(0)

    @pl.when(j == 0)
    def _init():
        e_scr[...] = _select_augment(e2_ref[...], p_ref[...])
        m_scr[...] = jnp.full((1, _BATCH), -1e30, jnp.float32)
        s_scr[...] = jnp.zeros((1, _BATCH), jnp.float32)

    x = lax.dot_general(
        w_ref[...], e_scr[...], (((0,), (1,)), ((), ())),
        preferred_element_type=jnp.float32,
    )
    row = jax.lax.broadcasted_iota(jnp.int32, (_VT, 1), 0) + j * _VT
    x = jnp.where(row < _VOCAB, x, -1e30)
    m_old = m_scr[...]
    m_new = jnp.maximum(m_old, jnp.max(x, axis=0, keepdims=True))
    s_scr[...] = s_scr[...] * jnp.exp(m_old - m_new) + jnp.sum(
        jnp.exp(x - m_new), axis=0, keepdims=True)
    m_scr[...] = m_new

    @pl.when(j == _NV - 1)
    def _fin():
        o_ref[...] = m_scr[...] + jnp.log(s_scr[...])


def _pass_a(e2, parity, w_aug):
    return pl.pallas_call(
        _lse_body,
        grid=(_NV,),
        in_specs=[
            pl.BlockSpec((_BATCH, 2 * _EMB), lambda j: (0, 0)),
            pl.BlockSpec((_BATCH, 1), lambda j: (0, 0)),
            pl.BlockSpec((_EMB + 1, _VT), lambda j: (0, j)),
        ],
        out_specs=pl.BlockSpec((1, _BATCH), lambda j: (0, 0)),
        out_shape=jax.ShapeDtypeStruct((1, _BATCH), jnp.float32),
        scratch_shapes=[
            pltpu.VMEM((_BATCH, _EMB + 1), jnp.bfloat16),
            pltpu.VMEM((1, _BATCH), jnp.float32),
            pltpu.VMEM((1, _BATCH), jnp.float32),
        ],
    )(e2, parity, w_aug)


def _write_body(e2_ref, p_ref, w_ref, l_ref, o_ref, e_scr):
    j = pl.program_id(0)

    @pl.when(j == 0)
    def _init():
        e_scr[...] = _select_augment(e2_ref[...], p_ref[...])

    x = lax.dot_general(
        w_ref[...], e_scr[...], (((0,), (1,)), ((), ())),
        preferred_element_type=jnp.float32,
    )
    o_ref[...] = x - l_ref[...]


def _pass_b(e2, parity, w_aug, lse):
    return pl.pallas_call(
        _write_body,
        grid=(_NV,),
        in_specs=[
            pl.BlockSpec((_BATCH, 2 * _EMB), lambda j: (0, 0)),
            pl.BlockSpec((_BATCH, 1), lambda j: (0, 0)),
            pl.BlockSpec((_EMB + 1, _VT), lambda j: (0, j)),
            pl.BlockSpec((1, _BATCH), lambda j: (0, 0)),
        ],
        out_specs=pl.BlockSpec((_VT, _BATCH), lambda j: (j, 0)),
        out_shape=jax.ShapeDtypeStruct((_VOCAB, _BATCH), jnp.float32),
        scratch_shapes=[
            pltpu.VMEM((_BATCH, _EMB + 1), jnp.bfloat16),
        ],
    )(e2, parity, w_aug, lse)


def kernel(input_word, emb_table, W, b):
    idx = input_word.astype(jnp.int32)
    table2 = emb_table.reshape(_VOCAB // 2, 2 * _EMB)
    e2 = _sc_gather_pairs(table2, idx >> 1)
    parity = (idx & 1).reshape(_BATCH, 1)
    w_aug = jnp.concatenate([W.T, b.reshape(1, _VOCAB)], axis=0)
    w_aug = w_aug.astype(jnp.bfloat16)
    lse = _pass_a(e2, parity, w_aug)
    out_t = _pass_b(e2, parity, w_aug, lse)
    return out_t.T


# speedup vs baseline: 1.8451x; 1.0177x over previous
import jax
import jax.numpy as jnp
from jax import lax
from jax.experimental import pallas as pl
from jax.experimental.pallas import tpu as pltpu
from jax.experimental.pallas import tpu_sc as plsc

_BATCH = 1024
_EMB = 64
_VOCAB = 100000

_NUM_WORKERS = 32
_ROWS_PER_WORKER = _BATCH // _NUM_WORKERS

_VT = 2048
_NV = pl.cdiv(_VOCAB, _VT)
_VTA = 4096
_NVA = pl.cdiv(_VOCAB, _VTA)


def _sc_gather_pairs(table2, idx_half):
    mesh = plsc.VectorSubcoreMesh(core_axis_name="c", subcore_axis_name="s")

    @pl.kernel(
        mesh=mesh,
        out_type=jax.ShapeDtypeStruct((_BATCH, 2 * _EMB), table2.dtype),
        scratch_types=[
            pltpu.VMEM((_ROWS_PER_WORKER,), jnp.int32),
            pltpu.VMEM((_ROWS_PER_WORKER, 2 * _EMB), table2.dtype),
            pltpu.SemaphoreType.DMA,
        ],
    )
    def gather_kernel(table_hbm, idx_hbm, out_hbm, idx_v, rows_v, sem):
        wid = lax.axis_index("s") * 2 + lax.axis_index("c")
        base = wid * _ROWS_PER_WORKER
        pltpu.sync_copy(idx_hbm.at[pl.ds(base, _ROWS_PER_WORKER)], idx_v)
        pltpu.async_copy(table_hbm.at[idx_v], rows_v, sem).wait()
        pltpu.sync_copy(rows_v, out_hbm.at[pl.ds(base, _ROWS_PER_WORKER)])

    return gather_kernel(table2, idx_half)


def _select_augment(e2, par):
    e = jnp.where(par == 1, e2[:, _EMB:], e2[:, :_EMB])
    ones = jnp.ones((_BATCH, 1), jnp.float32)
    return jnp.concatenate([e, ones], axis=1).astype(jnp.bfloat16)


def _online_update(x, m_scr, s_scr):
    m_old = m_scr[...]
    m_new = jnp.maximum(m_old, jnp.max(x, axis=0, keepdims=True))
    s_scr[...] = s_scr[...] * jnp.exp(m_old - m_new) + jnp.sum(
        jnp.exp(x - m_new), axis=0, keepdims=True)
    m_scr[...] = m_new


def _lse_body(e2_ref, p_ref, w_ref, o_ref, e_scr, m_scr, s_scr):
    j = pl.program_id(0)

    @pl.when(j == 0)
    def _init():
        e_scr[...] = _select_augment(e2_ref[...], p_ref[...])
        m_scr[...] = jnp.full((1, _BATCH), -1e30, jnp.float32)
        s_scr[...] = jnp.zeros((1, _BATCH), jnp.float32)

    x = lax.dot_general(
        w_ref[...], e_scr[...], (((0,), (1,)), ((), ())),
        preferred_element_type=jnp.float32,
    )

    @pl.when(j < _NVA - 1)
    def _full():
        _online_update(x, m_scr, s_scr)

    @pl.when(j == _NVA - 1)
    def _last():
        row = jax.lax.broadcasted_iota(jnp.int32, (_VTA, 1), 0) + j * _VTA
        _online_update(jnp.where(row < _VOCAB, x, -1e30), m_scr, s_scr)
        o_ref[...] = m_scr[...] + jnp.log(s_scr[...])


def _pass_a(e2, parity, w_aug):
    return pl.pallas_call(
        _lse_body,
        grid=(_NVA,),
        in_specs=[
            pl.BlockSpec((_BATCH, 2 * _EMB), lambda j: (0, 0)),
            pl.BlockSpec((_BATCH, 1), lambda j: (0, 0)),
            pl.BlockSpec((_EMB + 1, _VTA), lambda j: (0, j)),
        ],
        out_specs=pl.BlockSpec((1, _BATCH), lambda j: (0, 0)),
        out_shape=jax.ShapeDtypeStruct((1, _BATCH), jnp.float32),
        scratch_shapes=[
            pltpu.VMEM((_BATCH, _EMB + 1), jnp.bfloat16),
            pltpu.VMEM((1, _BATCH), jnp.float32),
            pltpu.VMEM((1, _BATCH), jnp.float32),
        ],
    )(e2, parity, w_aug)


def _write_body(e2_ref, p_ref, w_ref, l_ref, o_ref, e_scr):
    j = pl.program_id(0)

    @pl.when(j == 0)
    def _init():
        e_scr[...] = _select_augment(e2_ref[...], p_ref[...])

    x = lax.dot_general(
        w_ref[...], e_scr[...], (((0,), (1,)), ((), ())),
        preferred_element_type=jnp.float32,
    )
    o_ref[...] = x - l_ref[...]


def _pass_b(e2, parity, w_aug, lse):
    return pl.pallas_call(
        _write_body,
        grid=(_NV,),
        in_specs=[
            pl.BlockSpec((_BATCH, 2 * _EMB), lambda j: (0, 0)),
            pl.BlockSpec((_BATCH, 1), lambda j: (0, 0)),
            pl.BlockSpec((_EMB + 1, _VT), lambda j: (0, j)),
            pl.BlockSpec((1, _BATCH), lambda j: (0, 0)),
        ],
        out_specs=pl.BlockSpec((_VT, _BATCH), lambda j: (j, 0)),
        out_shape=jax.ShapeDtypeStruct((_VOCAB, _BATCH), jnp.float32),
        scratch_shapes=[
            pltpu.VMEM((_BATCH, _EMB + 1), jnp.bfloat16),
        ],
    )(e2, parity, w_aug, lse)


def kernel(input_word, emb_table, W, b):
    idx = input_word.astype(jnp.int32)
    table2 = emb_table.reshape(_VOCAB // 2, 2 * _EMB)
    e2 = _sc_gather_pairs(table2, idx >> 1)
    parity = (idx & 1).reshape(_BATCH, 1)
    w_aug = jnp.concatenate([W.T, b.reshape(1, _VOCAB)], axis=0)
    w_aug = w_aug.astype(jnp.bfloat16)
    lse = _pass_a(e2, parity, w_aug)
    out_t = _pass_b(e2, parity, w_aug, lse)
    return out_t.T
